# baseline (speedup 1.0000x reference)
import functools

import jax
import jax.numpy as jnp
from jax import lax
from jax.experimental import pallas as pl
from jax.experimental.pallas import tpu as pltpu
from jax.experimental.pallas import tpu_sc as plsc

NUM_ROWS = 16384 * 200
DIM = 64
N_WORKERS = 32
PER_W = NUM_ROWS // N_WORKERS
CHUNK = 128
IDX_STAGE = 6400
N_OUTER = PER_W // IDX_STAGE
N_INNER = IDX_STAGE // CHUNK

_mesh = plsc.VectorSubcoreMesh(core_axis_name="c", subcore_axis_name="s")


@functools.partial(
    pl.kernel,
    out_type=jax.ShapeDtypeStruct((NUM_ROWS, DIM), jnp.float32),
    mesh=_mesh,
    scratch_types=[
        pltpu.VMEM((IDX_STAGE,), jnp.int32),
        pltpu.VMEM((CHUNK, DIM), jnp.float32),
        pltpu.SemaphoreType.DMA,
    ],
    compiler_params=pltpu.CompilerParams(use_tc_tiling_on_sc=False),
)
def _embed_sc(city_hbm, table_hbm, out_hbm, idx_v, rows_v, sem):
    wid = lax.axis_index("s") * 2 + lax.axis_index("c")
    base = wid * PER_W

    @pl.loop(0, N_OUTER)
    def _outer(o):
        stage_base = base + o * IDX_STAGE
        pltpu.sync_copy(city_hbm.at[pl.ds(stage_base, IDX_STAGE)], idx_v)

        @pl.loop(0, N_INNER)
        def _inner(j):
            pltpu.async_copy(
                table_hbm.at[idx_v.at[pl.ds(j * CHUNK, CHUNK)]], rows_v, sem
            ).wait()
            pltpu.sync_copy(
                rows_v, out_hbm.at[pl.ds(stage_base + j * CHUNK, CHUNK)]
            )


def kernel(city, table):
    flat = jnp.reshape(city, (NUM_ROWS,)).astype(jnp.int32)
    out = _embed_sc(flat, table)
    return jnp.reshape(out, (*city.shape, DIM))


# speedup vs baseline: 5.9797x; 5.9797x over previous
import functools

import jax
import jax.numpy as jnp
from jax import lax
from jax.experimental import pallas as pl
from jax.experimental.pallas import tpu as pltpu
from jax.experimental.pallas import tpu_sc as plsc

NUM_ROWS = 16384 * 200
DIM = 64
TAB = 5
N_WORKERS = 32
PER_W = NUM_ROWS // N_WORKERS
CHUNK = 128
SUPER = 640
G = SUPER // CHUNK
IDX_STAGE = 6400
N_OUTER = PER_W // IDX_STAGE
NS = IDX_STAGE // SUPER

_mesh = plsc.VectorSubcoreMesh(core_axis_name="c", subcore_axis_name="s")


@functools.partial(
    pl.kernel,
    out_type=jax.ShapeDtypeStruct((NUM_ROWS, DIM), jnp.float32),
    mesh=_mesh,
    scratch_types=[
        pltpu.VMEM((2, IDX_STAGE), jnp.int32),
        pltpu.VMEM((2, SUPER, DIM), jnp.float32),
        pltpu.SemaphoreType.DMA,
        pltpu.SemaphoreType.DMA,
        pltpu.SemaphoreType.DMA,
    ],
    compiler_params=pltpu.CompilerParams(use_tc_tiling_on_sc=False),
)
def _embed_sc(city_hbm, table_hbm, out_hbm, idx2, rows2, isem, gsem, wsem):
    wid = lax.axis_index("s") * 2 + lax.axis_index("c")
    base = wid * PER_W
    tbase = wid * TAB

    def fire_stage(o, slot):
        pltpu.async_copy(
            city_hbm.at[pl.ds(base + o * IDX_STAGE, IDX_STAGE)],
            idx2.at[slot], isem,
        )

    def wait_stage(slot):
        pltpu.make_async_copy(
            city_hbm.at[pl.ds(0, IDX_STAGE)], idx2.at[slot], isem
        ).wait()

    def fire_gathers(slot, s, buf):
        for g in range(G):
            pltpu.async_copy(
                table_hbm.at[pl.ds(tbase, TAB)].at[
                    idx2.at[slot, pl.ds(s * SUPER + g * CHUNK, CHUNK)]
                ],
                rows2.at[buf, pl.ds(g * CHUNK, CHUNK)], gsem,
            )

    def drain_gathers(buf):
        for g in range(G):
            pltpu.make_async_copy(
                table_hbm.at[pl.ds(0, CHUNK)],
                rows2.at[buf, pl.ds(g * CHUNK, CHUNK)], gsem,
            ).wait()

    def fire_write(o, s, buf):
        pltpu.async_copy(
            rows2.at[buf],
            out_hbm.at[pl.ds(base + (o * NS + s) * SUPER, SUPER)], wsem,
        )

    def drain_write(buf):
        pltpu.make_async_copy(
            out_hbm.at[pl.ds(0, SUPER)], rows2.at[buf], wsem
        ).wait()

    def super_body(o, slot, s, buf, prime_next=True):
        drain_gathers(buf)
        fire_write(o, s, buf)
        @pl.when(o * NS + s > 0)
        def _():
            drain_write(1 - buf)
        if prime_next:
            fire_gathers(slot, s + 1, 1 - buf)

    def outer_body(o, slot):
        @pl.when(o + 1 < N_OUTER)
        def _():
            fire_stage(o + 1, 1 - slot)
        wait_stage(slot)
        fire_gathers(slot, 0, 0)
        @pl.loop(0, NS - 2, step=2)
        def _inner(s):
            super_body(o, slot, s, 0)
            super_body(o, slot, s + 1, 1)
        super_body(o, slot, NS - 2, 0)
        super_body(o, slot, NS - 1, 1, prime_next=False)

    fire_stage(0, 0)

    @pl.loop(0, N_OUTER, step=2)
    def _outer(o):
        outer_body(o, 0)
        outer_body(o + 1, 1)

    drain_write(1)


def kernel(city, table):
    flat = jnp.reshape(city, (NUM_ROWS,)).astype(jnp.int32)
    table_rep = jnp.tile(table, (N_WORKERS, 1))
    out = _embed_sc(flat, table_rep)
    return jnp.reshape(out, (*city.shape, DIM))


# speedup vs baseline: 5.9840x; 1.0007x over previous
import functools

import jax
import jax.numpy as jnp
from jax import lax
from jax.experimental import pallas as pl
from jax.experimental.pallas import tpu as pltpu
from jax.experimental.pallas import tpu_sc as plsc

NUM_ROWS = 16384 * 200
DIM = 64
TAB = 5
N_WORKERS = 32
PER_W = NUM_ROWS // N_WORKERS
CHUNK = 640
SUPER = 640
G = SUPER // CHUNK
IDX_STAGE = 6400
N_OUTER = PER_W // IDX_STAGE
NS = IDX_STAGE // SUPER

_mesh = plsc.VectorSubcoreMesh(core_axis_name="c", subcore_axis_name="s")


@functools.partial(
    pl.kernel,
    out_type=jax.ShapeDtypeStruct((NUM_ROWS, DIM), jnp.float32),
    mesh=_mesh,
    scratch_types=[
        pltpu.VMEM((2, IDX_STAGE), jnp.int32),
        pltpu.VMEM((2, SUPER, DIM), jnp.float32),
        pltpu.SemaphoreType.DMA,
        pltpu.SemaphoreType.DMA,
        pltpu.SemaphoreType.DMA,
    ],
    compiler_params=pltpu.CompilerParams(use_tc_tiling_on_sc=False),
)
def _embed_sc(city_hbm, table_hbm, out_hbm, idx2, rows2, isem, gsem, wsem):
    wid = lax.axis_index("s") * 2 + lax.axis_index("c")
    base = wid * PER_W
    tbase = wid * TAB

    def fire_stage(o, slot):
        pltpu.async_copy(
            city_hbm.at[pl.ds(base + o * IDX_STAGE, IDX_STAGE)],
            idx2.at[slot], isem,
        )

    def wait_stage(slot):
        pltpu.make_async_copy(
            city_hbm.at[pl.ds(0, IDX_STAGE)], idx2.at[slot], isem
        ).wait()

    def fire_gathers(slot, s, buf):
        for g in range(G):
            pltpu.async_copy(
                table_hbm.at[pl.ds(tbase, TAB)].at[
                    idx2.at[slot, pl.ds(s * SUPER + g * CHUNK, CHUNK)]
                ],
                rows2.at[buf, pl.ds(g * CHUNK, CHUNK)], gsem,
            )

    def drain_gathers(buf):
        for g in range(G):
            pltpu.make_async_copy(
                out_hbm.at[pl.ds(0, CHUNK)],
                rows2.at[buf, pl.ds(g * CHUNK, CHUNK)], gsem,
            ).wait()

    def fire_write(o, s, buf):
        pltpu.async_copy(
            rows2.at[buf],
            out_hbm.at[pl.ds(base + (o * NS + s) * SUPER, SUPER)], wsem,
        )

    def drain_write(buf):
        pltpu.make_async_copy(
            out_hbm.at[pl.ds(0, SUPER)], rows2.at[buf], wsem
        ).wait()

    def super_body(o, slot, s, buf, prime_next=True):
        drain_gathers(buf)
        fire_write(o, s, buf)
        @pl.when(o * NS + s > 0)
        def _():
            drain_write(1 - buf)
        if prime_next:
            fire_gathers(slot, s + 1, 1 - buf)

    def outer_body(o, slot):
        @pl.when(o + 1 < N_OUTER)
        def _():
            fire_stage(o + 1, 1 - slot)
        wait_stage(slot)
        fire_gathers(slot, 0, 0)
        @pl.loop(0, NS - 2, step=2)
        def _inner(s):
            super_body(o, slot, s, 0)
            super_body(o, slot, s + 1, 1)
        super_body(o, slot, NS - 2, 0)
        super_body(o, slot, NS - 1, 1, prime_next=False)

    fire_stage(0, 0)

    @pl.loop(0, N_OUTER, step=2)
    def _outer(o):
        outer_body(o, 0)
        outer_body(o + 1, 1)

    drain_write(1)


def kernel(city, table):
    flat = jnp.reshape(city, (NUM_ROWS,)).astype(jnp.int32)
    table_rep = jnp.tile(table, (N_WORKERS, 1))
    out = _embed_sc(flat, table_rep)
    return jnp.reshape(out, (*city.shape, DIM))
